# Initial kernel scaffold; baseline (speedup 1.0000x reference)
import functools

import jax
import jax.numpy as jnp
from jax import lax
from jax.experimental import pallas as pl
from jax.experimental.pallas import tpu as pltpu
from jax.experimental.pallas import tpu_sc as plsc

NUM_CORES = 2
NUM_SUBCORES = 16
NW = NUM_CORES * NUM_SUBCORES

EMBED_DIM = 64
K = 26


def _bag_sum_sc(idx4, table, n_bags, chunks_per_worker, chunk):
    bags_per_worker = n_bags // NW
    mesh = plsc.VectorSubcoreMesh(
        core_axis_name="c", subcore_axis_name="s",
        num_cores=NUM_CORES, num_subcores=NUM_SUBCORES)

    @functools.partial(
        pl.kernel,
        out_type=jax.ShapeDtypeStruct((n_bags, EMBED_DIM), jnp.float32),
        mesh=mesh,
        scratch_types=[
            pltpu.VMEM((K, chunk), jnp.int32),
            pltpu.VMEM((chunk, EMBED_DIM), jnp.float32),
            pltpu.SemaphoreType.DMA,
        ],
    )
    def k(idx_hbm, table_hbm, out_hbm, idx_v, acc_v, sem):
        wid = lax.axis_index("s") * NUM_CORES + lax.axis_index("c")
        base0 = wid * bags_per_worker

        @pl.loop(0, chunks_per_worker)
        def _chunk(c):
            pltpu.sync_copy(idx_hbm.at[wid, c], idx_v)
            pltpu.async_copy(table_hbm.at[idx_v.at[0]], acc_v, sem).wait()
            cps = [
                pltpu.async_copy(table_hbm.at[idx_v.at[j]], acc_v, sem,
                                 add=True)
                for j in range(1, K)
            ]
            for cp in cps:
                cp.wait()
            pltpu.sync_copy(acc_v, out_hbm.at[pl.ds(base0 + c * chunk, chunk)])

    return k(idx4, table)


def kernel(x, table):
    batch, seq, k = x.shape
    n_bags = batch * seq
    chunk = 80
    chunks_per_worker = n_bags // (NW * chunk)
    idx = x.reshape(n_bags, k).astype(jnp.int32)
    idx4 = idx.reshape(NW, chunks_per_worker, chunk, k).transpose(0, 1, 3, 2)
    out = _bag_sum_sc(idx4, table, n_bags, chunks_per_worker, chunk)
    return out.reshape(batch, seq, EMBED_DIM)


# speedup vs baseline: 2.4566x; 2.4566x over previous
import functools

import jax
import jax.numpy as jnp
from jax import lax
from jax.experimental import pallas as pl
from jax.experimental.pallas import tpu as pltpu
from jax.experimental.pallas import tpu_sc as plsc

NUM_CORES = 2
NUM_SUBCORES = 16
NW = NUM_CORES * NUM_SUBCORES

EMBED_DIM = 64
K = 26


def _bag_sum_sc(idx4, table, n_bags, chunks_per_worker, chunk):
    bags_per_worker = n_bags // NW
    mesh = plsc.VectorSubcoreMesh(
        core_axis_name="c", subcore_axis_name="s",
        num_cores=NUM_CORES, num_subcores=NUM_SUBCORES)

    @functools.partial(
        pl.kernel,
        out_type=jax.ShapeDtypeStruct((n_bags, EMBED_DIM), jnp.float32),
        mesh=mesh,
        scratch_types=[
            pltpu.VMEM((K, chunk), jnp.int32),
            pltpu.VMEM((chunk, EMBED_DIM), jnp.float32),
            pltpu.SemaphoreType.DMA,
        ],
        compiler_params=pltpu.CompilerParams(use_tc_tiling_on_sc=False),
    )
    def k(idx_hbm, table_hbm, out_hbm, idx_v, acc_v, sem):
        wid = lax.axis_index("s") * NUM_CORES + lax.axis_index("c")
        base0 = wid * bags_per_worker

        @pl.loop(0, chunks_per_worker)
        def _chunk(c):
            pltpu.sync_copy(idx_hbm.at[wid, c], idx_v)
            pltpu.async_copy(table_hbm.at[idx_v.at[0]], acc_v, sem).wait()
            cps = [
                pltpu.async_copy(table_hbm.at[idx_v.at[j]], acc_v, sem,
                                 add=True)
                for j in range(1, K)
            ]
            for cp in cps:
                cp.wait()
            pltpu.sync_copy(acc_v, out_hbm.at[pl.ds(base0 + c * chunk, chunk)])

    return k(idx4, table)


def kernel(x, table):
    batch, seq, k = x.shape
    n_bags = batch * seq
    chunk = 80
    chunks_per_worker = n_bags // (NW * chunk)
    idx = x.reshape(n_bags, k).astype(jnp.int32)
    idx4 = idx.reshape(NW, chunks_per_worker, chunk, k).transpose(0, 1, 3, 2)
    out = _bag_sum_sc(idx4, table, n_bags, chunks_per_worker, chunk)
    return out.reshape(batch, seq, EMBED_DIM)
